# baseline (speedup 1.0000x reference)
import jax
import jax.numpy as jnp
from jax import lax
from jax.experimental import pallas as pl
from jax.experimental.pallas import tpu as pltpu

_PS = 4
_CAP = 2


def _gate_body(K, P, T, D, E, p_ref, lns_ref, lnb_ref, cpW_ref, cpb_ref,
               glW_ref, glb_ref, posm_ref, wtn_ref):
    p = p_ref[0]
    pm = jnp.mean(p, axis=0)
    pv = jnp.mean((p - pm[None, :, :]) ** 2, axis=0)
    mu = jnp.mean(pm, axis=1, keepdims=True)
    var = jnp.mean((pm - mu) ** 2, axis=1, keepdims=True)
    ln = (pm - mu) / jnp.sqrt(var + 1e-5) * lns_ref[...] + lnb_ref[...]
    cpv = jnp.dot(pv, cpW_ref[...],
                  preferred_element_type=jnp.float32) + cpb_ref[...]
    logits = (jnp.dot(ln, glW_ref[0:D, :], preferred_element_type=jnp.float32)
              + jnp.dot(cpv, glW_ref[D:2 * D, :], preferred_element_type=jnp.float32)
              + glb_ref[...])
    mx = jnp.max(logits, axis=1, keepdims=True)
    ex = jnp.exp(logits - mx)
    aff = ex / jnp.sum(ex, axis=1, keepdims=True)
    affT = jnp.transpose(aff)
    a_t = affT[:, :, None]
    a_s = affT[:, None, :]
    i_t = lax.broadcasted_iota(jnp.int32, (E, T, T), 1)
    i_s = lax.broadcasted_iota(jnp.int32, (E, T, T), 2)
    beats = (a_s > a_t) | ((a_s == a_t) & (i_s < i_t))
    rank = jnp.sum(beats.astype(jnp.float32), axis=2)
    maskf = (rank < K).astype(jnp.float32)
    ii = lax.broadcasted_iota(jnp.int32, (T, T), 0)
    jj = lax.broadcasted_iota(jnp.int32, (T, T), 1)
    lt = (ii < jj).astype(jnp.float32)
    pos = jnp.dot(maskf, lt, preferred_element_type=jnp.float32)
    posm_ref[0] = jnp.where(maskf > 0, pos, -1.0).astype(jnp.int32)
    tot = jnp.sum(maskf * affT, axis=0, keepdims=True)
    wtn_ref[0] = maskf * affT / jnp.maximum(tot, 1e-8)


def _moe_body(K, P, T, D, E, B, pbf_ref, posm_ref, wtn_ref,
              Wg_ref, Wu_ref, Wd_ref, out_ref, psel_ref):
    e = pl.program_id(0)

    @pl.when(e == 0)
    def _init():
        out_ref[...] = jnp.zeros_like(out_ref)

    jrow = lax.broadcasted_iota(jnp.int32, (K, T), 0)
    Gws = []
    for b in range(B):
        posm_row = posm_ref[b, pl.ds(e, 1), :]
        wt_row = wtn_ref[b, pl.ds(e, 1), :]
        G = (jrow == posm_row)
        Gws.append(jnp.where(G, wt_row, 0.0))
        Gb = G.astype(jnp.bfloat16)
        for j in range(P):
            psel_ref[pl.ds((b * P + j) * K, K), :] = jnp.dot(
                Gb, pbf_ref[b, j],
                preferred_element_type=jnp.float32).astype(jnp.bfloat16)

    psel = psel_ref[...]
    hg = jnp.dot(psel, Wg_ref[0], preferred_element_type=jnp.float32)
    hu = jnp.dot(psel, Wu_ref[0], preferred_element_type=jnp.float32)
    h = (hg * lax.logistic(hg) * hu).astype(jnp.bfloat16)
    proc = jnp.dot(h, Wd_ref[0], preferred_element_type=jnp.float32)
    for b in range(B):
        for j in range(P):
            c = lax.dot_general(Gws[b], proc[(b * P + j) * K:(b * P + j + 1) * K, :],
                                (((0,), (0,)), ((), ())),
                                preferred_element_type=jnp.float32)
            out_ref[b, j] += c


def kernel(x, ln_scale, ln_bias, cp_W, cp_b, gl_W, gl_b, Wg, Wu, Wd):
    B, S, D = x.shape
    E = gl_W.shape[1]
    HW = int(round(S ** 0.5))
    Th = HW // _PS
    T = Th * Th
    P = _PS * _PS
    K = max(1, int(T / E * _CAP))

    x2d = x.reshape(B, HW, HW, D)
    p_tok = (x2d.reshape(B, Th, _PS, Th, _PS, D)
             .transpose(0, 2, 4, 1, 3, 5)
             .reshape(B, P, T, D))

    def gate_body(*refs):
        _gate_body(K, P, T, D, E, *refs)

    posm, wtn = pl.pallas_call(
        gate_body,
        grid=(B,),
        in_specs=[
            pl.BlockSpec((1, P, T, D), lambda b: (b, 0, 0, 0)),
            pl.BlockSpec((1, D), lambda b: (0, 0)),
            pl.BlockSpec((1, D), lambda b: (0, 0)),
            pl.BlockSpec((D, D), lambda b: (0, 0)),
            pl.BlockSpec((1, D), lambda b: (0, 0)),
            pl.BlockSpec((2 * D, E), lambda b: (0, 0)),
            pl.BlockSpec((1, E), lambda b: (0, 0)),
        ],
        out_specs=[
            pl.BlockSpec((1, E, T), lambda b: (b, 0, 0)),
            pl.BlockSpec((1, E, T), lambda b: (b, 0, 0)),
        ],
        out_shape=[
            jax.ShapeDtypeStruct((B, E, T), jnp.int32),
            jax.ShapeDtypeStruct((B, E, T), jnp.float32),
        ],
    )(p_tok, ln_scale.reshape(1, D), ln_bias.reshape(1, D), cp_W,
      cp_b.reshape(1, D), gl_W, gl_b.reshape(1, E))

    def moe_body(*refs):
        _moe_body(K, P, T, D, E, B, *refs)

    I = Wg.shape[2]
    out_tok = pl.pallas_call(
        moe_body,
        grid=(E,),
        in_specs=[
            pl.BlockSpec((B, P, T, D), lambda e: (0, 0, 0, 0)),
            pl.BlockSpec((B, E, T), lambda e: (0, 0, 0)),
            pl.BlockSpec((B, E, T), lambda e: (0, 0, 0)),
            pl.BlockSpec((1, D, I), lambda e: (e, 0, 0)),
            pl.BlockSpec((1, D, I), lambda e: (e, 0, 0)),
            pl.BlockSpec((1, I, D), lambda e: (e, 0, 0)),
        ],
        out_specs=pl.BlockSpec((B, P, T, D), lambda e: (0, 0, 0, 0)),
        out_shape=jax.ShapeDtypeStruct((B, P, T, D), jnp.float32),
        scratch_shapes=[
            pltpu.VMEM((B * P * K, D), jnp.bfloat16),
        ],
    )(p_tok.astype(jnp.bfloat16), posm, wtn,
      Wg.astype(jnp.bfloat16), Wu.astype(jnp.bfloat16),
      Wd.astype(jnp.bfloat16))

    out2d = (out_tok.reshape(B, _PS, _PS, Th, Th, D)
             .transpose(0, 3, 1, 4, 2, 5)
             .reshape(B, HW, HW, D))
    return out2d.reshape(B, S, D)


# speedup vs baseline: 1.1618x; 1.1618x over previous
import jax
import jax.numpy as jnp
from jax import lax
from jax.experimental import pallas as pl
from jax.experimental.pallas import tpu as pltpu

_PS = 4
_CAP = 2


def _body(K, P, T, D, E, B, Th, x_ref, lns_ref, lnb_ref, cpW_ref, cpb_ref,
          glW_ref, glb_ref, Wg_ref, Wu_ref, Wd_ref, out_ref,
          pbf_ref, posm_ref, wtn_ref, psel_ref, proc_ref):
    e = pl.program_id(0)

    @pl.when(e == 0)
    def _gate():
        for b in range(B):
            slabs = []
            for pr in range(_PS):
                for pc in range(_PS):
                    slab = x_ref[b, :, pr, :, pc, :].reshape(T, D)
                    slabs.append(slab)
                    pbf_ref[b, pr * _PS + pc] = slab.astype(jnp.bfloat16)
            pm = sum(slabs) * (1.0 / P)
            pv = sum((s - pm) ** 2 for s in slabs) * (1.0 / P)
            mu = jnp.mean(pm, axis=1, keepdims=True)
            var = jnp.mean((pm - mu) ** 2, axis=1, keepdims=True)
            ln = (pm - mu) / jnp.sqrt(var + 1e-5) * lns_ref[...] + lnb_ref[...]
            cpv = jnp.dot(pv, cpW_ref[...],
                          preferred_element_type=jnp.float32) + cpb_ref[...]
            logits = (jnp.dot(ln, glW_ref[0:D, :],
                              preferred_element_type=jnp.float32)
                      + jnp.dot(cpv, glW_ref[D:2 * D, :],
                                preferred_element_type=jnp.float32)
                      + glb_ref[...])
            mx = jnp.max(logits, axis=1, keepdims=True)
            ex = jnp.exp(logits - mx)
            aff = ex / jnp.sum(ex, axis=1, keepdims=True)
            affT = jnp.transpose(aff)
            a_t = affT[:, :, None]
            a_s = affT[:, None, :]
            i_t = lax.broadcasted_iota(jnp.int32, (E, T, T), 1)
            i_s = lax.broadcasted_iota(jnp.int32, (E, T, T), 2)
            beats = (a_s > a_t) | ((a_s == a_t) & (i_s < i_t))
            rank = jnp.sum(beats.astype(jnp.float32), axis=2)
            maskf = (rank < K).astype(jnp.float32)
            ii = lax.broadcasted_iota(jnp.int32, (T, T), 0)
            jj = lax.broadcasted_iota(jnp.int32, (T, T), 1)
            lt = (ii < jj).astype(jnp.float32)
            pos = jnp.dot(maskf, lt, preferred_element_type=jnp.float32)
            posm_ref[pl.ds(b * E, E), :] = jnp.where(
                maskf > 0, pos, -1.0).astype(jnp.int32)
            tot = jnp.sum(maskf * affT, axis=0, keepdims=True)
            wtn_ref[pl.ds(b * E, E), :] = maskf * affT / jnp.maximum(tot, 1e-8)

    jrow = lax.broadcasted_iota(jnp.int32, (K, T), 0)
    for b in range(B):
        posm_row = posm_ref[pl.ds(b * E + e, 1), :]
        Gb = (jrow == posm_row).astype(jnp.bfloat16)
        for j in range(P):
            psel_ref[pl.ds((b * P + j) * K, K), :] = jnp.dot(
                Gb, pbf_ref[b, j],
                preferred_element_type=jnp.float32).astype(jnp.bfloat16)

    M = B * P * K
    Q = M // 4
    Wgb = Wg_ref[0].astype(jnp.bfloat16)
    Wub = Wu_ref[0].astype(jnp.bfloat16)
    Wdb = Wd_ref[0].astype(jnp.bfloat16)
    for c0 in (0, Q, 2 * Q, 3 * Q):
        psel = psel_ref[pl.ds(c0, Q), :]
        hg = jnp.dot(psel, Wgb, preferred_element_type=jnp.float32)
        hu = jnp.dot(psel, Wub, preferred_element_type=jnp.float32)
        h = (hg * lax.logistic(hg) * hu).astype(jnp.bfloat16)
        proc = jnp.dot(h, Wdb,
                       preferred_element_type=jnp.float32).astype(jnp.bfloat16)
        proc_ref[pl.ds(e * M + c0, Q), :] = proc

    @pl.when(e == E - 1)
    def _combine():
        jmod3 = lax.broadcasted_iota(jnp.int32, (E, K, T), 1)
        for b in range(B):
            posm3 = posm_ref[pl.ds(b * E, E), :][:, None, :]
            wtn3 = wtn_ref[pl.ds(b * E, E), :][:, None, :]
            CbT = jnp.where(jmod3 == posm3, wtn3, 0.0)
            CbT = CbT.reshape(E * K, T).astype(jnp.bfloat16)
            for j in range(P):
                bj = b * P + j
                pa = jnp.concatenate(
                    [proc_ref[pl.ds(ee * M + bj * K, K), :]
                     for ee in range(E)], axis=0)
                val = lax.dot_general(
                    CbT, pa, (((0,), (0,)), ((), ())),
                    preferred_element_type=jnp.float32)
                pr, pc = j // _PS, j % _PS
                out_ref[b, :, pr, :, pc, :] = val.astype(
                    jnp.bfloat16).reshape(Th, Th, D)


def kernel(x, ln_scale, ln_bias, cp_W, cp_b, gl_W, gl_b, Wg, Wu, Wd):
    B, S, D = x.shape
    E = gl_W.shape[1]
    HW = int(round(S ** 0.5))
    Th = HW // _PS
    T = Th * Th
    P = _PS * _PS
    K = max(1, int(T / E * _CAP))
    I = Wg.shape[2]

    x6 = x.reshape(B, Th, _PS, Th, _PS, D)

    def body(*refs):
        _body(K, P, T, D, E, B, Th, *refs)

    out6 = pl.pallas_call(
        body,
        grid=(E,),
        in_specs=[
            pl.BlockSpec((B, Th, _PS, Th, _PS, D),
                         lambda e: (0, 0, 0, 0, 0, 0)),
            pl.BlockSpec((1, D), lambda e: (0, 0)),
            pl.BlockSpec((1, D), lambda e: (0, 0)),
            pl.BlockSpec((D, D), lambda e: (0, 0)),
            pl.BlockSpec((1, D), lambda e: (0, 0)),
            pl.BlockSpec((2 * D, E), lambda e: (0, 0)),
            pl.BlockSpec((1, E), lambda e: (0, 0)),
            pl.BlockSpec((1, D, I), lambda e: (e, 0, 0)),
            pl.BlockSpec((1, D, I), lambda e: (e, 0, 0)),
            pl.BlockSpec((1, I, D), lambda e: (e, 0, 0)),
        ],
        out_specs=pl.BlockSpec((B, Th, _PS, Th, _PS, D),
                               lambda e: (0, 0, 0, 0, 0, 0)),
        out_shape=jax.ShapeDtypeStruct((B, Th, _PS, Th, _PS, D),
                                       jnp.bfloat16),
        scratch_shapes=[
            pltpu.VMEM((B, P, T, D), jnp.bfloat16),
            pltpu.VMEM((B * E, T), jnp.int32),
            pltpu.VMEM((B * E, T), jnp.float32),
            pltpu.VMEM((B * P * K, D), jnp.bfloat16),
            pltpu.VMEM((E * B * P * K, D), jnp.bfloat16),
        ],
    )(x6, ln_scale.reshape(1, D), ln_bias.reshape(1, D), cp_W,
      cp_b.reshape(1, D), gl_W, gl_b.reshape(1, E), Wg, Wu, Wd)

    return out6.astype(jnp.float32).reshape(B, S, D)


# speedup vs baseline: 1.6026x; 1.3794x over previous
import jax
import jax.numpy as jnp
from jax import lax
from jax.experimental import pallas as pl
from jax.experimental.pallas import tpu as pltpu

_PS = 4
_CAP = 2
_KP = 48


def _gate_body(K, P, T, D, E, Th, x_ref, lns_ref, lnb_ref, cpW_ref, cpb_ref,
               glW_ref, glb_ref, posm_ref, wtn_ref, pbf_ref):
    slabs = []
    for pr in range(_PS):
        for pc in range(_PS):
            slab = x_ref[0, :, pr, :, pc, :].reshape(T, D)
            slabs.append(slab)
            pbf_ref[0, pr * _PS + pc] = slab.astype(jnp.bfloat16)
    pm = sum(slabs) * (1.0 / P)
    pv = sum((s - pm) ** 2 for s in slabs) * (1.0 / P)
    mu = jnp.mean(pm, axis=1, keepdims=True)
    var = jnp.mean((pm - mu) ** 2, axis=1, keepdims=True)
    ln = (pm - mu) / jnp.sqrt(var + 1e-5) * lns_ref[...] + lnb_ref[...]
    cpv = jnp.dot(pv, cpW_ref[...],
                  preferred_element_type=jnp.float32) + cpb_ref[...]
    logits = (jnp.dot(ln, glW_ref[0:D, :], preferred_element_type=jnp.float32)
              + jnp.dot(cpv, glW_ref[D:2 * D, :], preferred_element_type=jnp.float32)
              + glb_ref[...])
    mx = jnp.max(logits, axis=1, keepdims=True)
    ex = jnp.exp(logits - mx)
    aff = ex / jnp.sum(ex, axis=1, keepdims=True)
    affT = jnp.transpose(aff)
    a_t = affT[:, :, None]
    a_s = affT[:, None, :]
    i_t = lax.broadcasted_iota(jnp.int32, (E, T, T), 1)
    i_s = lax.broadcasted_iota(jnp.int32, (E, T, T), 2)
    beats = (a_s > a_t) | ((a_s == a_t) & (i_s < i_t))
    rank = jnp.sum(beats.astype(jnp.float32), axis=2)
    maskf = (rank < K).astype(jnp.float32)
    ii = lax.broadcasted_iota(jnp.int32, (T, T), 0)
    jj = lax.broadcasted_iota(jnp.int32, (T, T), 1)
    lt = (ii < jj).astype(jnp.float32)
    pos = jnp.dot(maskf, lt, preferred_element_type=jnp.float32)
    posm_ref[0] = jnp.where(maskf > 0, pos, -1.0).astype(jnp.int32)
    tot = jnp.sum(maskf * affT, axis=0, keepdims=True)
    wtn_ref[0] = maskf * affT / jnp.maximum(tot, 1e-8)


def _moe_body(K, P, T, D, E, B, Th, pbf_ref, posm_ref, wtn_ref,
              Wg_ref, Wu_ref, Wd_ref, out_ref, psel_ref, proc_ref):
    e = pl.program_id(0)
    jrow = lax.broadcasted_iota(jnp.int32, (K, T), 0)

    for b in range(B):
        posm_row = posm_ref[b, pl.ds(e, 1), :]
        Gb = (jrow == posm_row).astype(jnp.bfloat16)
        for j in range(P):
            psel_ref[pl.ds((b * P + j) * K, K), :] = jnp.dot(
                Gb, pbf_ref[b, j],
                preferred_element_type=jnp.float32).astype(jnp.bfloat16)

    M = B * P * K
    half = M // 2
    zpad = jnp.zeros((_KP - K, D), jnp.bfloat16)
    Wgb = Wg_ref[0].astype(jnp.bfloat16)
    Wub = Wu_ref[0].astype(jnp.bfloat16)
    Wdb = Wd_ref[0].astype(jnp.bfloat16)
    for c0 in (0, half):
        psel = psel_ref[pl.ds(c0, half), :]
        hg = jnp.dot(psel, Wgb, preferred_element_type=jnp.float32)
        hu = jnp.dot(psel, Wub, preferred_element_type=jnp.float32)
        h = (hg * lax.logistic(hg) * hu).astype(jnp.bfloat16)
        proc = jnp.dot(h, Wdb,
                       preferred_element_type=jnp.float32).astype(jnp.bfloat16)
        n0 = c0 // K
        for i in range(half // K):
            bj = n0 + i
            piece = jnp.concatenate([proc[i * K:(i + 1) * K, :], zpad], axis=0)
            proc_ref[pl.ds(bj * E * _KP + e * _KP, _KP), :] = piece

    @pl.when(e == E - 1)
    def _combine():
        jmod3 = lax.broadcasted_iota(jnp.int32, (E, _KP, T), 1)
        for b in range(B):
            posm3 = posm_ref[b][:, None, :]
            wtn3 = wtn_ref[b][:, None, :]
            CbT = jnp.where(jmod3 == posm3, wtn3, 0.0)
            CbT = CbT.reshape(E * _KP, T).astype(jnp.bfloat16)
            for j in range(P):
                pa = proc_ref[(b * P + j) * E * _KP:(b * P + j + 1) * E * _KP, :]
                val = lax.dot_general(
                    CbT, pa, (((0,), (0,)), ((), ())),
                    preferred_element_type=jnp.float32)
                pr, pc = j // _PS, j % _PS
                out_ref[b, :, pr, :, pc, :] = val.reshape(Th, Th, D)


def kernel(x, ln_scale, ln_bias, cp_W, cp_b, gl_W, gl_b, Wg, Wu, Wd):
    B, S, D = x.shape
    E = gl_W.shape[1]
    HW = int(round(S ** 0.5))
    Th = HW // _PS
    T = Th * Th
    P = _PS * _PS
    K = max(1, int(T / E * _CAP))

    x6 = x.reshape(B, Th, _PS, Th, _PS, D)

    def gate_body(*refs):
        _gate_body(K, P, T, D, E, Th, *refs)

    posm, wtn, pbf = pl.pallas_call(
        gate_body,
        grid=(B,),
        in_specs=[
            pl.BlockSpec((1, Th, _PS, Th, _PS, D),
                         lambda b: (b, 0, 0, 0, 0, 0)),
            pl.BlockSpec((1, D), lambda b: (0, 0)),
            pl.BlockSpec((1, D), lambda b: (0, 0)),
            pl.BlockSpec((D, D), lambda b: (0, 0)),
            pl.BlockSpec((1, D), lambda b: (0, 0)),
            pl.BlockSpec((2 * D, E), lambda b: (0, 0)),
            pl.BlockSpec((1, E), lambda b: (0, 0)),
        ],
        out_specs=[
            pl.BlockSpec((1, E, T), lambda b: (b, 0, 0)),
            pl.BlockSpec((1, E, T), lambda b: (b, 0, 0)),
            pl.BlockSpec((1, P, T, D), lambda b: (b, 0, 0, 0)),
        ],
        out_shape=[
            jax.ShapeDtypeStruct((B, E, T), jnp.int32),
            jax.ShapeDtypeStruct((B, E, T), jnp.float32),
            jax.ShapeDtypeStruct((B, P, T, D), jnp.bfloat16),
        ],
    )(x6, ln_scale.reshape(1, D), ln_bias.reshape(1, D), cp_W,
      cp_b.reshape(1, D), gl_W, gl_b.reshape(1, E))

    def moe_body(*refs):
        _moe_body(K, P, T, D, E, B, Th, *refs)

    I = Wg.shape[2]
    out6 = pl.pallas_call(
        moe_body,
        grid=(E,),
        in_specs=[
            pl.BlockSpec((B, P, T, D), lambda e: (0, 0, 0, 0)),
            pl.BlockSpec((B, E, T), lambda e: (0, 0, 0)),
            pl.BlockSpec((B, E, T), lambda e: (0, 0, 0)),
            pl.BlockSpec((1, D, I), lambda e: (e, 0, 0)),
            pl.BlockSpec((1, D, I), lambda e: (e, 0, 0)),
            pl.BlockSpec((1, I, D), lambda e: (e, 0, 0)),
        ],
        out_specs=pl.BlockSpec((B, Th, _PS, Th, _PS, D),
                               lambda e: (0, 0, 0, 0, 0, 0)),
        out_shape=jax.ShapeDtypeStruct((B, Th, _PS, Th, _PS, D), jnp.float32),
        scratch_shapes=[
            pltpu.VMEM((B * P * K, D), jnp.bfloat16),
            pltpu.VMEM((B * P * E * _KP, D), jnp.bfloat16),
        ],
    )(pbf, posm, wtn, Wg, Wu, Wd)

    return out6.reshape(B, S, D)
